# Initial kernel scaffold; baseline (speedup 1.0000x reference)
import functools

import jax
import jax.numpy as jnp
from jax import lax
from jax.experimental import pallas as pl
from jax.experimental.pallas import tpu as pltpu
from jax.experimental.pallas import tpu_sc as plsc

N = 10000
E = 320000
D = 128
ED = 16

NP = 10240
NTILES = 32
EPT = E // NTILES
C = 80
NCHUNK = EPT // C
RPT = NP // 16
RSTAGE = 128

f32 = jnp.float32


def _mm2_body(x_ref, wl_ref, wr_ref, ol_ref, or_ref):
    x = x_ref[...]
    ol_ref[...] = jnp.dot(x, wl_ref[...], preferred_element_type=f32)
    or_ref[...] = jnp.dot(x, wr_ref[...], preferred_element_type=f32)


def _node_mm(x, Wl, Wr):
    blk = 1000
    return pl.pallas_call(
        _mm2_body,
        grid=(N // blk,),
        in_specs=[
            pl.BlockSpec((blk, D), lambda i: (i, 0)),
            pl.BlockSpec((D, D), lambda i: (0, 0)),
            pl.BlockSpec((D, D), lambda i: (0, 0)),
        ],
        out_specs=[pl.BlockSpec((blk, D), lambda i: (i, 0))] * 2,
        out_shape=[jax.ShapeDtypeStruct((N, D), f32)] * 2,
    )(x, Wl, Wr)


def _mm1_body(a_ref, w_ref, o_ref):
    o_ref[...] = jnp.dot(a_ref[...], w_ref[...], preferred_element_type=f32)


def _edge_mm(ea, We):
    blk = 8000
    return pl.pallas_call(
        _mm1_body,
        grid=(E // blk,),
        in_specs=[
            pl.BlockSpec((blk, ED), lambda i: (i, 0)),
            pl.BlockSpec((ED, D), lambda i: (0, 0)),
        ],
        out_specs=pl.BlockSpec((blk, D), lambda i: (i, 0)),
        out_shape=jax.ShapeDtypeStruct((E, D), f32),
    )(ea, We)


def _fin_body(num_ref, den_ref, b_ref, o_ref, *, relu):
    num = num_ref[0, :N, :] + num_ref[1, :N, :]
    den = den_ref[0, :N] + den_ref[1, :N]
    o = num / (den[:, None] + 1e-16) + b_ref[...]
    if relu:
        o = jnp.maximum(o, 0.0)
    o_ref[...] = o


def _finalize(num, den, b, relu):
    return pl.pallas_call(
        functools.partial(_fin_body, relu=relu),
        in_specs=[
            pl.BlockSpec((2, NP, D), lambda: (0, 0, 0)),
            pl.BlockSpec((2, NP), lambda: (0, 0)),
            pl.BlockSpec((1, D), lambda: (0, 0)),
        ],
        out_specs=pl.BlockSpec((N, D), lambda: (0, 0)),
        out_shape=jax.ShapeDtypeStruct((N, D), f32),
    )(num, den, b.reshape(1, D))


def _sc_edge_body(xl_hbm, xr_hbm, ew_hbm, src_hbm, dst_hbm, att_hbm,
                  num_hbm, den_hbm,
                  src_idx, dst_idx, xl_rows, xr_rows, ew_rows,
                  a_buf, p_buf, att_v, zrow, zden,
                  num_acc, den_acc, sem0, sem1, sem2):
    cid = lax.axis_index("c")
    sid = lax.axis_index("s")
    gid = cid * 16 + sid

    pltpu.sync_copy(att_hbm, att_v)

    def _zero_zrow(i, _):
        r = i // 8
        c = i % 8
        zrow[r, pl.ds(c * 16, 16)] = jnp.zeros((16,), f32)
        return 0
    lax.fori_loop(0, RSTAGE * 8, _zero_zrow, 0)

    def _zero_zden(i, _):
        zden[pl.ds(i * 16, 16)] = jnp.zeros((16,), f32)
        return 0
    lax.fori_loop(0, RPT // 16, _zero_zden, 0)

    row0 = sid * RPT
    def _zero_acc(i, _):
        pltpu.sync_copy(zrow, num_acc.at[pl.ds(row0 + i * RSTAGE, RSTAGE), :])
        return 0
    lax.fori_loop(0, RPT // RSTAGE, _zero_acc, 0)
    pltpu.sync_copy(zden, den_acc.at[pl.ds(row0, RPT)])
    plsc.subcore_barrier()

    ebase = gid * EPT

    def _chunk(k, _):
        e0 = ebase + k * C
        pltpu.sync_copy(src_hbm.at[pl.ds(e0, C)], src_idx)
        pltpu.sync_copy(dst_hbm.at[pl.ds(e0, C)], dst_idx)
        d_xl = pltpu.async_copy(xl_hbm.at[src_idx], xl_rows, sem0)
        d_xr = pltpu.async_copy(xr_hbm.at[dst_idx], xr_rows, sem1)
        d_ew = pltpu.async_copy(ew_hbm.at[pl.ds(e0, C)], ew_rows, sem2)
        d_xl.wait()
        d_xr.wait()
        d_ew.wait()

        def _edge_a(i, _):
            acc = jnp.zeros((16,), f32)
            for c in range(8):
                sl = pl.ds(c * 16, 16)
                e = xl_rows[i, sl] + xr_rows[i, sl] + ew_rows[i, sl]
                e = jnp.where(e > 0.0, e, 0.2 * e)
                acc = acc + e * att_v[sl]
            a_buf[i] = jnp.sum(acc)
            return 0
        lax.fori_loop(0, C, _edge_a, 0)

        def _expv(j, _):
            sl = pl.ds(j * 16, 16)
            p_buf[sl] = jnp.exp(a_buf[sl])
            return 0
        lax.fori_loop(0, C // 16, _expv, 0)

        def _edge_s(i, _):
            pv = jnp.full((16,), p_buf[i], f32)
            for c in range(8):
                sl = pl.ds(c * 16, 16)
                xl_rows[i, sl] = xl_rows[i, sl] * pv
            return 0
        lax.fori_loop(0, C, _edge_s, 0)

        pltpu.sync_copy(xl_rows, num_acc.at[dst_idx], add=True)
        pltpu.sync_copy(p_buf, den_acc.at[dst_idx], add=True)
        return 0

    lax.fori_loop(0, NCHUNK, _chunk, 0)
    plsc.subcore_barrier()

    def _out(i, _):
        r = row0 + i * RSTAGE
        pltpu.sync_copy(num_acc.at[pl.ds(r, RSTAGE), :], zrow)
        pltpu.sync_copy(zrow, num_hbm.at[cid, pl.ds(r, RSTAGE), :])
        return 0
    lax.fori_loop(0, RPT // RSTAGE, _out, 0)
    pltpu.sync_copy(den_acc.at[pl.ds(row0, RPT)], zden)
    pltpu.sync_copy(zden, den_hbm.at[cid, pl.ds(row0, RPT)])


def _sc_edge_pass(xl, xr, ew, src, dst, att):
    mesh = plsc.VectorSubcoreMesh(core_axis_name="c", subcore_axis_name="s")
    kfn = pl.kernel(
        _sc_edge_body,
        out_type=(
            jax.ShapeDtypeStruct((2, NP, D), f32),
            jax.ShapeDtypeStruct((2, NP), f32),
        ),
        mesh=mesh,
        scratch_types=[
            pltpu.VMEM((C,), jnp.int32),
            pltpu.VMEM((C,), jnp.int32),
            pltpu.VMEM((C, D), f32),
            pltpu.VMEM((C, D), f32),
            pltpu.VMEM((C, D), f32),
            pltpu.VMEM((C,), f32),
            pltpu.VMEM((C,), f32),
            pltpu.VMEM((D,), f32),
            pltpu.VMEM((RSTAGE, D), f32),
            pltpu.VMEM((RPT,), f32),
            pltpu.VMEM_SHARED((NP, D), f32),
            pltpu.VMEM_SHARED((NP,), f32),
            pltpu.SemaphoreType.DMA,
            pltpu.SemaphoreType.DMA,
            pltpu.SemaphoreType.DMA,
        ],
    )
    return kfn(xl, xr, ew, src, dst, att)


def _layer(h, src, dst, ea, Wl, Wr, We, att, b, relu):
    xl, xr = _node_mm(h, Wl, Wr)
    ew = _edge_mm(ea, We)
    num, den = _sc_edge_pass(xl, xr, ew, src, dst, att)
    return _finalize(num, den, b, relu)


def kernel(x, edge_index, edge_attr,
           Wl1, Wr1, We1, att1, b1,
           Wl2, Wr2, We2, att2, b2,
           Wl3, Wr3, We3, att3, b3):
    src = edge_index[0]
    dst = edge_index[1]
    h = _layer(x, src, dst, edge_attr, Wl1, Wr1, We1, att1, b1, True)
    h = _layer(h, src, dst, edge_attr, Wl2, Wr2, We2, att2, b2, True)
    h = _layer(h, src, dst, edge_attr, Wl3, Wr3, We3, att3, b3, False)
    return h


# speedup vs baseline: 9.1927x; 9.1927x over previous
import functools

import jax
import jax.numpy as jnp
from jax import lax
from jax.experimental import pallas as pl
from jax.experimental.pallas import tpu as pltpu
from jax.experimental.pallas import tpu_sc as plsc

N = 10000
E = 320000
D = 128
ED = 16

NP = 10240
NTILES = 32
EPT = E // NTILES
C = 80
NCHUNK = EPT // C
RPT = NP // 16
RSTAGE = 128

f32 = jnp.float32


def _mm2_body(x_ref, wl_ref, wr_ref, ol_ref, or_ref):
    x = x_ref[...]
    ol_ref[...] = jnp.dot(x, wl_ref[...], preferred_element_type=f32)
    or_ref[...] = jnp.dot(x, wr_ref[...], preferred_element_type=f32)


def _node_mm(x, Wl, Wr):
    blk = 1000
    return pl.pallas_call(
        _mm2_body,
        grid=(N // blk,),
        in_specs=[
            pl.BlockSpec((blk, D), lambda i: (i, 0)),
            pl.BlockSpec((D, D), lambda i: (0, 0)),
            pl.BlockSpec((D, D), lambda i: (0, 0)),
        ],
        out_specs=[pl.BlockSpec((blk, D), lambda i: (i, 0))] * 2,
        out_shape=[jax.ShapeDtypeStruct((N, D), f32)] * 2,
    )(x, Wl, Wr)


def _mm1_body(a_ref, w_ref, o_ref):
    o_ref[...] = jnp.dot(a_ref[...], w_ref[...], preferred_element_type=f32)


def _edge_mm(ea, We):
    blk = 8000
    return pl.pallas_call(
        _mm1_body,
        grid=(E // blk,),
        in_specs=[
            pl.BlockSpec((blk, ED), lambda i: (i, 0)),
            pl.BlockSpec((ED, D), lambda i: (0, 0)),
        ],
        out_specs=pl.BlockSpec((blk, D), lambda i: (i, 0)),
        out_shape=jax.ShapeDtypeStruct((E, D), f32),
    )(ea, We)


def _fin_body(num_ref, den_ref, b_ref, o_ref, *, relu):
    num = num_ref[0, :N, :] + num_ref[1, :N, :]
    den = den_ref[0, :N] + den_ref[1, :N]
    o = num / (den[:, None] + 1e-16) + b_ref[...]
    if relu:
        o = jnp.maximum(o, 0.0)
    o_ref[...] = o


def _finalize(num, den, b, relu):
    return pl.pallas_call(
        functools.partial(_fin_body, relu=relu),
        in_specs=[
            pl.BlockSpec((2, NP, D), lambda: (0, 0, 0)),
            pl.BlockSpec((2, NP), lambda: (0, 0)),
            pl.BlockSpec((1, D), lambda: (0, 0)),
        ],
        out_specs=pl.BlockSpec((N, D), lambda: (0, 0)),
        out_shape=jax.ShapeDtypeStruct((N, D), f32),
    )(num, den, b.reshape(1, D))


def _sc_edge_body(xl_hbm, xr_hbm, ew_hbm, src_hbm, dst_hbm, att_hbm,
                  num_hbm, den_hbm,
                  src_idx, dst_idx, xl_rows, xr_rows, ew_rows,
                  a_buf, p_buf, att_v, zrow, zden,
                  num_acc, den_acc, sem0, sem1, sem2):
    cid = lax.axis_index("c")
    sid = lax.axis_index("s")
    gid = cid * 16 + sid

    pltpu.sync_copy(att_hbm, att_v)

    def _zero_zrow(i, _):
        r = i // 8
        c = i % 8
        zrow[r, pl.ds(c * 16, 16)] = jnp.zeros((16,), f32)
        return 0
    lax.fori_loop(0, RSTAGE * 8, _zero_zrow, 0)

    def _zero_zden(i, _):
        zden[pl.ds(i * 16, 16)] = jnp.zeros((16,), f32)
        return 0
    lax.fori_loop(0, RPT // 16, _zero_zden, 0)

    row0 = sid * RPT
    def _zero_acc(i, _):
        pltpu.sync_copy(zrow, num_acc.at[pl.ds(row0 + i * RSTAGE, RSTAGE), :])
        return 0
    lax.fori_loop(0, RPT // RSTAGE, _zero_acc, 0)
    pltpu.sync_copy(zden, den_acc.at[pl.ds(row0, RPT)])
    plsc.subcore_barrier()

    ebase = gid * EPT

    def _chunk(k, _):
        e0 = ebase + k * C
        pltpu.sync_copy(src_hbm.at[pl.ds(e0, C)], src_idx)
        pltpu.sync_copy(dst_hbm.at[pl.ds(e0, C)], dst_idx)
        d_xl = pltpu.async_copy(xl_hbm.at[src_idx], xl_rows, sem0)
        d_xr = pltpu.async_copy(xr_hbm.at[dst_idx], xr_rows, sem1)
        d_ew = pltpu.async_copy(ew_hbm.at[pl.ds(e0, C)], ew_rows, sem2)
        d_xl.wait()
        d_xr.wait()
        d_ew.wait()

        def _edge_a(i, _):
            acc = jnp.zeros((16,), f32)
            for c in range(8):
                sl = pl.ds(c * 16, 16)
                e = xl_rows[i, sl] + xr_rows[i, sl] + ew_rows[i, sl]
                e = jnp.where(e > 0.0, e, 0.2 * e)
                acc = acc + e * att_v[sl]
            a_buf[i] = jnp.sum(acc)
            return 0
        lax.fori_loop(0, C, _edge_a, 0)

        lane = lax.iota(jnp.int32, 16)

        def _expv(j, _):
            av = jnp.zeros((16,), f32)
            for l in range(16):
                av = jnp.where(lane == l, a_buf[j * 16 + l], av)
            p_buf[pl.ds(j * 16, 16)] = jnp.exp(av)
            return 0
        lax.fori_loop(0, C // 16, _expv, 0)

        def _edge_s(j, _):
            pv16 = p_buf[pl.ds(j * 16, 16)]
            for l in range(16):
                i = j * 16 + l
                pv = jnp.full((16,), pv16[l], f32)
                for c in range(8):
                    sl = pl.ds(c * 16, 16)
                    xl_rows[i, sl] = xl_rows[i, sl] * pv
            return 0
        lax.fori_loop(0, C // 16, _edge_s, 0)

        pltpu.sync_copy(xl_rows, num_acc.at[dst_idx], add=True)
        pltpu.sync_copy(p_buf, den_acc.at[dst_idx], add=True)
        return 0

    lax.fori_loop(0, NCHUNK, _chunk, 0)
    plsc.subcore_barrier()

    def _out(i, _):
        r = row0 + i * RSTAGE
        pltpu.sync_copy(num_acc.at[pl.ds(r, RSTAGE), :], zrow)
        pltpu.sync_copy(zrow, num_hbm.at[cid, pl.ds(r, RSTAGE), :])
        return 0
    lax.fori_loop(0, RPT // RSTAGE, _out, 0)
    pltpu.sync_copy(den_acc.at[pl.ds(row0, RPT)], zden)
    pltpu.sync_copy(zden, den_hbm.at[cid, pl.ds(row0, RPT)])


def _sc_edge_pass(xl, xr, ew, src, dst, att):
    mesh = plsc.VectorSubcoreMesh(core_axis_name="c", subcore_axis_name="s")
    kfn = pl.kernel(
        _sc_edge_body,
        out_type=(
            jax.ShapeDtypeStruct((2, NP, D), f32),
            jax.ShapeDtypeStruct((2, NP), f32),
        ),
        mesh=mesh,
        compiler_params=pltpu.CompilerParams(needs_layout_passes=False),
        scratch_types=[
            pltpu.VMEM((C,), jnp.int32),
            pltpu.VMEM((C,), jnp.int32),
            pltpu.VMEM((C, D), f32),
            pltpu.VMEM((C, D), f32),
            pltpu.VMEM((C, D), f32),
            pltpu.SMEM((C,), f32),
            pltpu.VMEM((C,), f32),
            pltpu.VMEM((D,), f32),
            pltpu.VMEM((RSTAGE, D), f32),
            pltpu.VMEM((RPT,), f32),
            pltpu.VMEM_SHARED((NP, D), f32),
            pltpu.VMEM_SHARED((NP,), f32),
            pltpu.SemaphoreType.DMA,
            pltpu.SemaphoreType.DMA,
            pltpu.SemaphoreType.DMA,
        ],
    )
    return kfn(xl, xr, ew, src, dst, att)


def _layer(h, src, dst, ea, Wl, Wr, We, att, b, relu):
    xl, xr = _node_mm(h, Wl, Wr)
    ew = _edge_mm(ea, We)
    num, den = _sc_edge_pass(xl, xr, ew, src, dst, att)
    return _finalize(num, den, b, relu)


def kernel(x, edge_index, edge_attr,
           Wl1, Wr1, We1, att1, b1,
           Wl2, Wr2, We2, att2, b2,
           Wl3, Wr3, We3, att3, b3):
    src = edge_index[0]
    dst = edge_index[1]
    h = _layer(x, src, dst, edge_attr, Wl1, Wr1, We1, att1, b1, True)
    h = _layer(h, src, dst, edge_attr, Wl2, Wr2, We2, att2, b2, True)
    h = _layer(h, src, dst, edge_attr, Wl3, Wr3, We3, att3, b3, False)
    return h


# speedup vs baseline: 14.1462x; 1.5389x over previous
import functools

import jax
import jax.numpy as jnp
from jax import lax
from jax.experimental import pallas as pl
from jax.experimental.pallas import tpu as pltpu
from jax.experimental.pallas import tpu_sc as plsc

N = 10000
E = 320000
D = 128
ED = 16

NP = 10240
NTILES = 32
EPT = E // NTILES
C = 48
NCHUNK = 208
REM = EPT - NCHUNK * C
RPT = NP // 16
SR = 40

f32 = jnp.float32


def _mm2_body(x_ref, wl_ref, wr_ref, ol_ref, or_ref):
    x = x_ref[...]
    ol_ref[...] = jnp.dot(x, wl_ref[...], preferred_element_type=f32)
    or_ref[...] = jnp.dot(x, wr_ref[...], preferred_element_type=f32)


def _node_mm(x, Wl, Wr):
    blk = 1000
    return pl.pallas_call(
        _mm2_body,
        grid=(N // blk,),
        in_specs=[
            pl.BlockSpec((blk, D), lambda i: (i, 0)),
            pl.BlockSpec((D, D), lambda i: (0, 0)),
            pl.BlockSpec((D, D), lambda i: (0, 0)),
        ],
        out_specs=[pl.BlockSpec((blk, D), lambda i: (i, 0))] * 2,
        out_shape=[jax.ShapeDtypeStruct((N, D), f32)] * 2,
    )(x, Wl, Wr)


def _mm1_body(a_ref, w_ref, o_ref):
    o_ref[...] = jnp.dot(a_ref[...], w_ref[...], preferred_element_type=f32)


def _edge_mm(ea, We):
    blk = 8000
    return pl.pallas_call(
        _mm1_body,
        grid=(E // blk,),
        in_specs=[
            pl.BlockSpec((blk, ED), lambda i: (i, 0)),
            pl.BlockSpec((ED, D), lambda i: (0, 0)),
        ],
        out_specs=pl.BlockSpec((blk, D), lambda i: (i, 0)),
        out_shape=jax.ShapeDtypeStruct((E, D), f32),
    )(ea, We)


def _fin_body(num_ref, den_ref, b_ref, o_ref, *, relu):
    num = num_ref[0, :N, :] + num_ref[1, :N, :]
    den = den_ref[0, :N] + den_ref[1, :N]
    o = num / (den[:, None] + 1e-16) + b_ref[...]
    if relu:
        o = jnp.maximum(o, 0.0)
    o_ref[...] = o


def _finalize(num, den, b, relu):
    return pl.pallas_call(
        functools.partial(_fin_body, relu=relu),
        in_specs=[
            pl.BlockSpec((2, NP, D), lambda: (0, 0, 0)),
            pl.BlockSpec((2, NP), lambda: (0, 0)),
            pl.BlockSpec((1, D), lambda: (0, 0)),
        ],
        out_specs=pl.BlockSpec((N, D), lambda: (0, 0)),
        out_shape=jax.ShapeDtypeStruct((N, D), f32),
    )(num, den, b.reshape(1, D))


def _sc_edge_body(xl_hbm, xr_hbm, ew_hbm, src_hbm, dst_hbm, att_hbm,
                  num_hbm, den_hbm,
                  si0, si1, di0, di1, dsc0, dsc1,
                  xl0, xl1, xr0, xr1, ew0, ew1,
                  a_buf, p_buf, att_v, zden,
                  num_acc, den_acc,
                  sem_si0, sem_si1, sem_di0, sem_di1, sem_xl0, sem_xl1,
                  sem_xr0, sem_xr1, sem_ew0, sem_ew1):
    cid = lax.axis_index("c")
    sid = lax.axis_index("s")
    gid = cid * 16 + sid

    sis = (si0, si1)
    dis = (di0, di1)
    dscs = (dsc0, dsc1)
    xls = (xl0, xl1)
    xrs = (xr0, xr1)
    ews = (ew0, ew1)
    sems_si = (sem_si0, sem_si1)
    sems_di = (sem_di0, sem_di1)
    sems_xl = (sem_xl0, sem_xl1)
    sems_xr = (sem_xr0, sem_xr1)
    sems_ew = (sem_ew0, sem_ew1)

    pltpu.sync_copy(att_hbm, att_v)

    def _zero_xl0(i, _):
        r = i // 8
        c = i % 8
        xl0[r, pl.ds(c * 16, 16)] = jnp.zeros((16,), f32)
        return 0
    lax.fori_loop(0, C * 8, _zero_xl0, 0)

    def _zero_zden(i, _):
        zden[pl.ds(i * 16, 16)] = jnp.zeros((16,), f32)
        return 0
    lax.fori_loop(0, RPT // 16, _zero_zden, 0)

    row0 = sid * RPT
    def _zero_acc(i, _):
        pltpu.sync_copy(xl0.at[pl.ds(0, SR), :],
                        num_acc.at[pl.ds(row0 + i * SR, SR), :])
        return 0
    lax.fori_loop(0, RPT // SR, _zero_acc, 0)
    pltpu.sync_copy(zden, den_acc.at[pl.ds(row0, RPT)])
    plsc.subcore_barrier()

    ebase = gid * EPT

    def _issue_idx(b, k):
        e0 = ebase + k * C
        pltpu.async_copy(src_hbm.at[pl.ds(e0, C)], sis[b], sems_si[b])
        pltpu.async_copy(dst_hbm.at[pl.ds(e0, C)], dis[b], sems_di[b])

    def _wait_idx(b):
        pltpu.make_async_copy(src_hbm.at[pl.ds(0, C)], sis[b],
                              sems_si[b]).wait()
        pltpu.make_async_copy(dst_hbm.at[pl.ds(0, C)], dis[b],
                              sems_di[b]).wait()

    def _issue_gathers(b, k):
        pltpu.async_copy(xl_hbm.at[sis[b]], xls[b], sems_xl[b])
        pltpu.async_copy(xr_hbm.at[dis[b]], xrs[b], sems_xr[b])
        pltpu.async_copy(ew_hbm.at[pl.ds(ebase + k * C, C)], ews[b],
                         sems_ew[b])

    def _wait_gathers(b):
        pltpu.make_async_copy(xl_hbm.at[sis[b]], xls[b],
                              sems_xl[b]).wait()
        pltpu.make_async_copy(xr_hbm.at[dis[b]], xrs[b],
                              sems_xr[b]).wait()
        pltpu.make_async_copy(ew_hbm.at[pl.ds(0, C)], ews[b],
                              sems_ew[b]).wait()

    def _save_dst(b, cnt=C):
        for j in range(cnt // 16):
            dscs[b][pl.ds(j * 16, 16)] = dis[b][pl.ds(j * 16, 16)]

    def _compute_scatter(b, cnt=C):
        xl_rows, xr_rows, ew_rows = xls[b], xrs[b], ews[b]

        att_c = tuple(att_v[pl.ds(c * 16, 16)] for c in range(8))
        lane = lax.iota(jnp.int32, 16)

        def _alpha_of(i):
            parts = []
            for c in range(8):
                sl = pl.ds(c * 16, 16)
                e = xl_rows[i, sl] + xr_rows[i, sl] + ew_rows[i, sl]
                e = jnp.maximum(e, 0.2 * e)
                parts.append(e * att_c[c])
            while len(parts) > 1:
                parts = [parts[2 * j] + parts[2 * j + 1]
                         for j in range(len(parts) // 2)]
            return jnp.sum(parts[0])

        def _edge_a(i2, carry):
            i = i2 * 2
            a_buf[i] = _alpha_of(i)
            a_buf[i + 1] = _alpha_of(i + 1)
            return carry
        lax.fori_loop(0, cnt // 2, _edge_a, 0)

        def _group(j, _):
            av = jnp.zeros((16,), f32)
            for l in range(16):
                av = jnp.where(lane == l, a_buf[j * 16 + l], av)
            pv16 = jnp.exp(av)
            p_buf[pl.ds(j * 16, 16)] = pv16
            for l in range(16):
                i = j * 16 + l
                pv = jnp.full((16,), pv16[l], f32)
                for c in range(8):
                    sl = pl.ds(c * 16, 16)
                    xl_rows[i, sl] = xl_rows[i, sl] * pv
            return 0
        lax.fori_loop(0, cnt // 16, _group, 0)

        if cnt == C:
            pltpu.sync_copy(xl_rows, num_acc.at[dscs[b]], add=True)
            pltpu.sync_copy(p_buf, den_acc.at[dscs[b]], add=True)
        else:
            pltpu.sync_copy(xl_rows.at[pl.ds(0, cnt), :],
                            num_acc.at[dscs[b].at[pl.ds(0, cnt)]], add=True)
            pltpu.sync_copy(p_buf.at[pl.ds(0, cnt)],
                            den_acc.at[dscs[b].at[pl.ds(0, cnt)]], add=True)

    _issue_idx(0, 0)
    _wait_idx(0)
    _issue_gathers(0, 0)
    _issue_idx(1, 1)

    def _pair(k2, _):
        for par in (0, 1):
            k = k2 * 2 + par
            nb = 1 - par
            _wait_idx(nb)
            _issue_gathers(nb, k + 1)
            _wait_gathers(par)
            _save_dst(par)
            @pl.when(k + 2 < NCHUNK)
            def _():
                _issue_idx(par, k + 2)
            _compute_scatter(par)
        return 0

    lax.fori_loop(0, (NCHUNK - 2) // 2, _pair, 0)
    _wait_idx(1)
    _issue_gathers(1, NCHUNK - 1)
    _wait_gathers(0)
    _save_dst(0)
    _compute_scatter(0)
    _wait_gathers(1)
    _save_dst(1)
    _compute_scatter(1)

    e0 = ebase + NCHUNK * C
    pltpu.sync_copy(src_hbm.at[pl.ds(e0, REM)], si0.at[pl.ds(0, REM)])
    pltpu.sync_copy(dst_hbm.at[pl.ds(e0, REM)], di0.at[pl.ds(0, REM)])
    pltpu.async_copy(xl_hbm.at[si0.at[pl.ds(0, REM)]],
                     xl0.at[pl.ds(0, REM), :], sem_xl0).wait()
    pltpu.async_copy(xr_hbm.at[di0.at[pl.ds(0, REM)]],
                     xr0.at[pl.ds(0, REM), :], sem_xr0).wait()
    pltpu.async_copy(ew_hbm.at[pl.ds(e0, REM)],
                     ew0.at[pl.ds(0, REM), :], sem_ew0).wait()
    _save_dst(0, REM)
    _compute_scatter(0, REM)
    plsc.subcore_barrier()

    def _out(i, _):
        r = row0 + i * SR
        pltpu.sync_copy(num_acc.at[pl.ds(r, SR), :], xl0.at[pl.ds(0, SR), :])
        pltpu.sync_copy(xl0.at[pl.ds(0, SR), :], num_hbm.at[cid, pl.ds(r, SR), :])
        return 0
    lax.fori_loop(0, RPT // SR, _out, 0)
    pltpu.sync_copy(den_acc.at[pl.ds(row0, RPT)], zden)
    pltpu.sync_copy(zden, den_hbm.at[cid, pl.ds(row0, RPT)])


def _sc_edge_pass(xl, xr, ew, src, dst, att):
    mesh = plsc.VectorSubcoreMesh(core_axis_name="c", subcore_axis_name="s")
    kfn = pl.kernel(
        _sc_edge_body,
        out_type=(
            jax.ShapeDtypeStruct((2, NP, D), f32),
            jax.ShapeDtypeStruct((2, NP), f32),
        ),
        mesh=mesh,
        compiler_params=pltpu.CompilerParams(needs_layout_passes=False),
        scratch_types=[
            pltpu.VMEM((C,), jnp.int32),
            pltpu.VMEM((C,), jnp.int32),
            pltpu.VMEM((C,), jnp.int32),
            pltpu.VMEM((C,), jnp.int32),
            pltpu.VMEM((C,), jnp.int32),
            pltpu.VMEM((C,), jnp.int32),
            pltpu.VMEM((C, D), f32),
            pltpu.VMEM((C, D), f32),
            pltpu.VMEM((C, D), f32),
            pltpu.VMEM((C, D), f32),
            pltpu.VMEM((C, D), f32),
            pltpu.VMEM((C, D), f32),
            pltpu.SMEM((C,), f32),
            pltpu.VMEM((C,), f32),
            pltpu.VMEM((D,), f32),
            pltpu.VMEM((RPT,), f32),
            pltpu.VMEM_SHARED((NP, D), f32),
            pltpu.VMEM_SHARED((NP,), f32),
            pltpu.SemaphoreType.DMA,
            pltpu.SemaphoreType.DMA,
            pltpu.SemaphoreType.DMA,
            pltpu.SemaphoreType.DMA,
            pltpu.SemaphoreType.DMA,
            pltpu.SemaphoreType.DMA,
            pltpu.SemaphoreType.DMA,
            pltpu.SemaphoreType.DMA,
            pltpu.SemaphoreType.DMA,
            pltpu.SemaphoreType.DMA,
        ],
    )
    return kfn(xl, xr, ew, src, dst, att)


def _layer(h, src, dst, ea, Wl, Wr, We, att, b, relu):
    xl, xr = _node_mm(h, Wl, Wr)
    ew = _edge_mm(ea, We)
    num, den = _sc_edge_pass(xl, xr, ew, src, dst, att)
    return _finalize(num, den, b, relu)


def kernel(x, edge_index, edge_attr,
           Wl1, Wr1, We1, att1, b1,
           Wl2, Wr2, We2, att2, b2,
           Wl3, Wr3, We3, att3, b3):
    src = edge_index[0]
    dst = edge_index[1]
    h = _layer(x, src, dst, edge_attr, Wl1, Wr1, We1, att1, b1, True)
    h = _layer(h, src, dst, edge_attr, Wl2, Wr2, We2, att2, b2, True)
    h = _layer(h, src, dst, edge_attr, Wl3, Wr3, We3, att3, b3, False)
    return h


# speedup vs baseline: 14.4939x; 1.0246x over previous
import functools

import jax
import jax.numpy as jnp
from jax import lax
from jax.experimental import pallas as pl
from jax.experimental.pallas import tpu as pltpu
from jax.experimental.pallas import tpu_sc as plsc

N = 10000
E = 320000
D = 128
ED = 16

NP = 10240
NTILES = 32
EPT = E // NTILES
C = 48
NCHUNK = 208
REM = EPT - NCHUNK * C
RPT = NP // 16
SR = 40

f32 = jnp.float32


def _mm2_body(x_ref, wl_ref, wr_ref, ol_ref, or_ref):
    x = x_ref[...]
    ol_ref[...] = jnp.dot(x, wl_ref[...], preferred_element_type=f32)
    or_ref[...] = jnp.dot(x, wr_ref[...], preferred_element_type=f32)


def _node_mm(x, Wl, Wr):
    blk = 1000
    return pl.pallas_call(
        _mm2_body,
        grid=(N // blk,),
        in_specs=[
            pl.BlockSpec((blk, D), lambda i: (i, 0)),
            pl.BlockSpec((D, D), lambda i: (0, 0)),
            pl.BlockSpec((D, D), lambda i: (0, 0)),
        ],
        out_specs=[pl.BlockSpec((blk, D), lambda i: (i, 0))] * 2,
        out_shape=[jax.ShapeDtypeStruct((N, D), f32)] * 2,
    )(x, Wl, Wr)


def _mm1_body(a_ref, w_ref, o_ref):
    o_ref[...] = jnp.dot(a_ref[...], w_ref[...], preferred_element_type=f32)


def _edge_mm(ea, We):
    blk = 8000
    return pl.pallas_call(
        _mm1_body,
        grid=(E // blk,),
        in_specs=[
            pl.BlockSpec((blk, ED), lambda i: (i, 0)),
            pl.BlockSpec((ED, D), lambda i: (0, 0)),
        ],
        out_specs=pl.BlockSpec((blk, D), lambda i: (i, 0)),
        out_shape=jax.ShapeDtypeStruct((E, D), f32),
    )(ea, We)


def _fin_body(num_ref, den_ref, b_ref, o_ref, *, relu):
    num = num_ref[0, :N, :] + num_ref[1, :N, :]
    den = den_ref[0, :N] + den_ref[1, :N]
    o = num / (den[:, None] + 1e-16) + b_ref[...]
    if relu:
        o = jnp.maximum(o, 0.0)
    o_ref[...] = o


def _finalize(num, den, b, relu):
    return pl.pallas_call(
        functools.partial(_fin_body, relu=relu),
        in_specs=[
            pl.BlockSpec((2, NP, D), lambda: (0, 0, 0)),
            pl.BlockSpec((2, NP), lambda: (0, 0)),
            pl.BlockSpec((1, D), lambda: (0, 0)),
        ],
        out_specs=pl.BlockSpec((N, D), lambda: (0, 0)),
        out_shape=jax.ShapeDtypeStruct((N, D), f32),
    )(num, den, b.reshape(1, D))


def _sc_edge_body(xl_hbm, xr_hbm, ew_hbm, src_hbm, dst_hbm, att_hbm,
                  num_hbm, den_hbm,
                  si0, si1, di0, di1, dsc0, dsc1,
                  xl0, xl1, xr0, xr1, ew0, ew1,
                  a_buf, p0, p1, att_v, zden,
                  num_acc, den_acc,
                  sem_si0, sem_si1, sem_di0, sem_di1, sem_xl0, sem_xl1,
                  sem_xr0, sem_xr1, sem_ew0, sem_ew1,
                  sem_sn0, sem_sn1, sem_sp0, sem_sp1):
    cid = lax.axis_index("c")
    sid = lax.axis_index("s")
    gid = cid * 16 + sid

    sis = (si0, si1)
    dis = (di0, di1)
    dscs = (dsc0, dsc1)
    pbs = (p0, p1)
    sems_sn = (sem_sn0, sem_sn1)
    sems_sp = (sem_sp0, sem_sp1)
    xls = (xl0, xl1)
    xrs = (xr0, xr1)
    ews = (ew0, ew1)
    sems_si = (sem_si0, sem_si1)
    sems_di = (sem_di0, sem_di1)
    sems_xl = (sem_xl0, sem_xl1)
    sems_xr = (sem_xr0, sem_xr1)
    sems_ew = (sem_ew0, sem_ew1)

    pltpu.sync_copy(att_hbm, att_v)

    def _zero_xl0(i, _):
        r = i // 8
        c = i % 8
        xl0[r, pl.ds(c * 16, 16)] = jnp.zeros((16,), f32)
        return 0
    lax.fori_loop(0, C * 8, _zero_xl0, 0)

    def _zero_zden(i, _):
        zden[pl.ds(i * 16, 16)] = jnp.zeros((16,), f32)
        return 0
    lax.fori_loop(0, RPT // 16, _zero_zden, 0)

    row0 = sid * RPT
    def _zero_acc(i, _):
        pltpu.sync_copy(xl0.at[pl.ds(0, SR), :],
                        num_acc.at[pl.ds(row0 + i * SR, SR), :])
        return 0
    lax.fori_loop(0, RPT // SR, _zero_acc, 0)
    pltpu.sync_copy(zden, den_acc.at[pl.ds(row0, RPT)])
    plsc.subcore_barrier()

    ebase = gid * EPT

    def _issue_idx(b, k):
        e0 = ebase + k * C
        pltpu.async_copy(src_hbm.at[pl.ds(e0, C)], sis[b], sems_si[b])
        pltpu.async_copy(dst_hbm.at[pl.ds(e0, C)], dis[b], sems_di[b])

    def _wait_idx(b):
        pltpu.make_async_copy(src_hbm.at[pl.ds(0, C)], sis[b],
                              sems_si[b]).wait()
        pltpu.make_async_copy(dst_hbm.at[pl.ds(0, C)], dis[b],
                              sems_di[b]).wait()

    def _issue_gathers(b, k):
        pltpu.async_copy(xl_hbm.at[sis[b]], xls[b], sems_xl[b])
        pltpu.async_copy(xr_hbm.at[dis[b]], xrs[b], sems_xr[b])
        pltpu.async_copy(ew_hbm.at[pl.ds(ebase + k * C, C)], ews[b],
                         sems_ew[b])

    def _wait_gathers(b):
        pltpu.make_async_copy(xl_hbm.at[sis[b]], xls[b],
                              sems_xl[b]).wait()
        pltpu.make_async_copy(xr_hbm.at[dis[b]], xrs[b],
                              sems_xr[b]).wait()
        pltpu.make_async_copy(ew_hbm.at[pl.ds(0, C)], ews[b],
                              sems_ew[b]).wait()

    def _save_dst(b, cnt=C):
        for j in range(cnt // 16):
            dscs[b][pl.ds(j * 16, 16)] = dis[b][pl.ds(j * 16, 16)]

    def _wait_scatters(b):
        pltpu.make_async_copy(xls[b], num_acc.at[dscs[b]],
                              sems_sn[b]).wait()
        pltpu.make_async_copy(pbs[b], den_acc.at[dscs[b]],
                              sems_sp[b]).wait()

    def _wait_scatters_rem(b, cnt):
        pltpu.make_async_copy(xls[b].at[pl.ds(0, cnt), :],
                              num_acc.at[dscs[b].at[pl.ds(0, cnt)]],
                              sems_sn[b]).wait()
        pltpu.make_async_copy(pbs[b].at[pl.ds(0, cnt)],
                              den_acc.at[dscs[b].at[pl.ds(0, cnt)]],
                              sems_sp[b]).wait()

    def _compute_scatter(b, cnt=C):
        xl_rows, xr_rows, ew_rows = xls[b], xrs[b], ews[b]

        att_c = tuple(att_v[pl.ds(c * 16, 16)] for c in range(8))
        lane = lax.iota(jnp.int32, 16)

        def _alpha_of(i):
            parts = []
            for c in range(8):
                sl = pl.ds(c * 16, 16)
                e = xl_rows[i, sl] + xr_rows[i, sl] + ew_rows[i, sl]
                e = jnp.maximum(e, 0.2 * e)
                parts.append(e * att_c[c])
            while len(parts) > 1:
                parts = [parts[2 * j] + parts[2 * j + 1]
                         for j in range(len(parts) // 2)]
            return jnp.sum(parts[0])

        def _edge_a(i2, carry):
            i = i2 * 2
            a_buf[i] = _alpha_of(i)
            a_buf[i + 1] = _alpha_of(i + 1)
            return carry
        lax.fori_loop(0, cnt // 2, _edge_a, 0)

        def _group(j, _):
            av = jnp.zeros((16,), f32)
            for l in range(16):
                av = jnp.where(lane == l, a_buf[j * 16 + l], av)
            pv16 = jnp.exp(av)
            pbs[b][pl.ds(j * 16, 16)] = pv16
            for l in range(16):
                i = j * 16 + l
                pv = jnp.full((16,), pv16[l], f32)
                for c in range(8):
                    sl = pl.ds(c * 16, 16)
                    xl_rows[i, sl] = xl_rows[i, sl] * pv
            return 0
        lax.fori_loop(0, cnt // 16, _group, 0)

        if cnt == C:
            pltpu.async_copy(xl_rows, num_acc.at[dscs[b]], sems_sn[b],
                             add=True)
            pltpu.async_copy(pbs[b], den_acc.at[dscs[b]], sems_sp[b],
                             add=True)
        else:
            pltpu.async_copy(xl_rows.at[pl.ds(0, cnt), :],
                             num_acc.at[dscs[b].at[pl.ds(0, cnt)]],
                             sems_sn[b], add=True)
            pltpu.async_copy(pbs[b].at[pl.ds(0, cnt)],
                             den_acc.at[dscs[b].at[pl.ds(0, cnt)]],
                             sems_sp[b], add=True)

    _issue_idx(0, 0)
    _wait_idx(0)
    _issue_gathers(0, 0)
    _issue_idx(1, 1)

    def _pair(k2, _):
        for par in (0, 1):
            k = k2 * 2 + par
            nb = 1 - par
            _wait_idx(nb)
            @pl.when(k >= 1)
            def _():
                _wait_scatters(nb)
            _issue_gathers(nb, k + 1)
            _wait_gathers(par)
            _save_dst(par)
            @pl.when(k + 2 < NCHUNK)
            def _():
                _issue_idx(par, k + 2)
            _compute_scatter(par)
        return 0

    lax.fori_loop(0, (NCHUNK - 2) // 2, _pair, 0)
    _wait_idx(1)
    _wait_scatters(1)
    _issue_gathers(1, NCHUNK - 1)
    _wait_gathers(0)
    _save_dst(0)
    _compute_scatter(0)
    _wait_gathers(1)
    _save_dst(1)
    _compute_scatter(1)

    _wait_scatters(0)
    e0 = ebase + NCHUNK * C
    pltpu.sync_copy(src_hbm.at[pl.ds(e0, REM)], si0.at[pl.ds(0, REM)])
    pltpu.sync_copy(dst_hbm.at[pl.ds(e0, REM)], di0.at[pl.ds(0, REM)])
    pltpu.async_copy(xl_hbm.at[si0.at[pl.ds(0, REM)]],
                     xl0.at[pl.ds(0, REM), :], sem_xl0).wait()
    pltpu.async_copy(xr_hbm.at[di0.at[pl.ds(0, REM)]],
                     xr0.at[pl.ds(0, REM), :], sem_xr0).wait()
    pltpu.async_copy(ew_hbm.at[pl.ds(e0, REM)],
                     ew0.at[pl.ds(0, REM), :], sem_ew0).wait()
    _save_dst(0, REM)
    _compute_scatter(0, REM)
    _wait_scatters_rem(0, REM)
    _wait_scatters(1)
    plsc.subcore_barrier()

    def _out(i, _):
        r = row0 + i * SR
        pltpu.sync_copy(num_acc.at[pl.ds(r, SR), :], xl0.at[pl.ds(0, SR), :])
        pltpu.sync_copy(xl0.at[pl.ds(0, SR), :], num_hbm.at[cid, pl.ds(r, SR), :])
        return 0
    lax.fori_loop(0, RPT // SR, _out, 0)
    pltpu.sync_copy(den_acc.at[pl.ds(row0, RPT)], zden)
    pltpu.sync_copy(zden, den_hbm.at[cid, pl.ds(row0, RPT)])


def _sc_edge_pass(xl, xr, ew, src, dst, att):
    mesh = plsc.VectorSubcoreMesh(core_axis_name="c", subcore_axis_name="s")
    kfn = pl.kernel(
        _sc_edge_body,
        out_type=(
            jax.ShapeDtypeStruct((2, NP, D), f32),
            jax.ShapeDtypeStruct((2, NP), f32),
        ),
        mesh=mesh,
        compiler_params=pltpu.CompilerParams(needs_layout_passes=False),
        scratch_types=[
            pltpu.VMEM((C,), jnp.int32),
            pltpu.VMEM((C,), jnp.int32),
            pltpu.VMEM((C,), jnp.int32),
            pltpu.VMEM((C,), jnp.int32),
            pltpu.VMEM((C,), jnp.int32),
            pltpu.VMEM((C,), jnp.int32),
            pltpu.VMEM((C, D), f32),
            pltpu.VMEM((C, D), f32),
            pltpu.VMEM((C, D), f32),
            pltpu.VMEM((C, D), f32),
            pltpu.VMEM((C, D), f32),
            pltpu.VMEM((C, D), f32),
            pltpu.SMEM((C,), f32),
            pltpu.VMEM((C,), f32),
            pltpu.VMEM((C,), f32),
            pltpu.VMEM((D,), f32),
            pltpu.VMEM((RPT,), f32),
            pltpu.VMEM_SHARED((NP, D), f32),
            pltpu.VMEM_SHARED((NP,), f32),
            pltpu.SemaphoreType.DMA,
            pltpu.SemaphoreType.DMA,
            pltpu.SemaphoreType.DMA,
            pltpu.SemaphoreType.DMA,
            pltpu.SemaphoreType.DMA,
            pltpu.SemaphoreType.DMA,
            pltpu.SemaphoreType.DMA,
            pltpu.SemaphoreType.DMA,
            pltpu.SemaphoreType.DMA,
            pltpu.SemaphoreType.DMA,
            pltpu.SemaphoreType.DMA,
            pltpu.SemaphoreType.DMA,
            pltpu.SemaphoreType.DMA,
            pltpu.SemaphoreType.DMA,
        ],
    )
    return kfn(xl, xr, ew, src, dst, att)


def _layer(h, src, dst, ea, Wl, Wr, We, att, b, relu):
    xl, xr = _node_mm(h, Wl, Wr)
    ew = _edge_mm(ea, We)
    num, den = _sc_edge_pass(xl, xr, ew, src, dst, att)
    return _finalize(num, den, b, relu)


def kernel(x, edge_index, edge_attr,
           Wl1, Wr1, We1, att1, b1,
           Wl2, Wr2, We2, att2, b2,
           Wl3, Wr3, We3, att3, b3):
    src = edge_index[0]
    dst = edge_index[1]
    h = _layer(x, src, dst, edge_attr, Wl1, Wr1, We1, att1, b1, True)
    h = _layer(h, src, dst, edge_attr, Wl2, Wr2, We2, att2, b2, True)
    h = _layer(h, src, dst, edge_attr, Wl3, Wr3, We3, att3, b3, False)
    return h


# speedup vs baseline: 14.7002x; 1.0142x over previous
import functools

import jax
import jax.numpy as jnp
from jax import lax
from jax.experimental import pallas as pl
from jax.experimental.pallas import tpu as pltpu
from jax.experimental.pallas import tpu_sc as plsc

N = 10000
E = 320000
D = 128
ED = 16

NP = 10240
NTILES = 32
EPT = E // NTILES
C = 48
NCHUNK = 208
REM = EPT - NCHUNK * C
RPT = NP // 16
SR = 40

f32 = jnp.float32


def _mm2_body(x_ref, wl_ref, wr_ref, ol_ref, or_ref):
    x = x_ref[...]
    ol_ref[...] = jnp.dot(x, wl_ref[...], preferred_element_type=f32)
    or_ref[...] = jnp.dot(x, wr_ref[...], preferred_element_type=f32)


def _node_mm(x, Wl, Wr):
    blk = 1000
    return pl.pallas_call(
        _mm2_body,
        grid=(N // blk,),
        in_specs=[
            pl.BlockSpec((blk, D), lambda i: (i, 0)),
            pl.BlockSpec((D, D), lambda i: (0, 0)),
            pl.BlockSpec((D, D), lambda i: (0, 0)),
        ],
        out_specs=[pl.BlockSpec((blk, D), lambda i: (i, 0))] * 2,
        out_shape=[jax.ShapeDtypeStruct((N, D), f32)] * 2,
    )(x, Wl, Wr)


def _finmm_body(num_ref, den_ref, b_ref, wl_ref, wr_ref, ol_ref, or_ref):
    num = num_ref[0, :N, :] + num_ref[1, :N, :]
    den = den_ref[0, :N] + den_ref[1, :N]
    h = num / (den[:, None] + 1e-16) + b_ref[...]
    h = jnp.maximum(h, 0.0)
    ol_ref[...] = jnp.dot(h, wl_ref[...], preferred_element_type=f32)
    or_ref[...] = jnp.dot(h, wr_ref[...], preferred_element_type=f32)


def _fin_mm(num, den, b, Wl, Wr):
    return pl.pallas_call(
        _finmm_body,
        in_specs=[
            pl.BlockSpec((2, NP, D), lambda: (0, 0, 0)),
            pl.BlockSpec((2, NP), lambda: (0, 0)),
            pl.BlockSpec((1, D), lambda: (0, 0)),
            pl.BlockSpec((D, D), lambda: (0, 0)),
            pl.BlockSpec((D, D), lambda: (0, 0)),
        ],
        out_specs=[pl.BlockSpec((N, D), lambda: (0, 0))] * 2,
        out_shape=[jax.ShapeDtypeStruct((N, D), f32)] * 2,
    )(num, den, b.reshape(1, D), Wl, Wr)


def _mm1_body(a_ref, w_ref, o_ref):
    o_ref[...] = jnp.dot(a_ref[...], w_ref[...], preferred_element_type=f32)


def _edge_mm(ea, We):
    blk = 8000
    return pl.pallas_call(
        _mm1_body,
        grid=(E // blk,),
        in_specs=[
            pl.BlockSpec((blk, ED), lambda i: (i, 0)),
            pl.BlockSpec((ED, D), lambda i: (0, 0)),
        ],
        out_specs=pl.BlockSpec((blk, D), lambda i: (i, 0)),
        out_shape=jax.ShapeDtypeStruct((E, D), f32),
    )(ea, We)


def _fin_body(num_ref, den_ref, b_ref, o_ref, *, relu):
    num = num_ref[0, :N, :] + num_ref[1, :N, :]
    den = den_ref[0, :N] + den_ref[1, :N]
    o = num / (den[:, None] + 1e-16) + b_ref[...]
    if relu:
        o = jnp.maximum(o, 0.0)
    o_ref[...] = o


def _finalize(num, den, b, relu):
    return pl.pallas_call(
        functools.partial(_fin_body, relu=relu),
        in_specs=[
            pl.BlockSpec((2, NP, D), lambda: (0, 0, 0)),
            pl.BlockSpec((2, NP), lambda: (0, 0)),
            pl.BlockSpec((1, D), lambda: (0, 0)),
        ],
        out_specs=pl.BlockSpec((N, D), lambda: (0, 0)),
        out_shape=jax.ShapeDtypeStruct((N, D), f32),
    )(num, den, b.reshape(1, D))


def _sc_edge_body(xl_hbm, xr_hbm, ew_hbm, src_hbm, dst_hbm, att_hbm,
                  num_hbm, den_hbm,
                  si0, si1, di0, di1, dsc0, dsc1,
                  xl0, xl1, xr0, xr1, ew0, ew1,
                  a_buf, p0, p1, att_v, zden,
                  num_acc, den_acc,
                  sem_si0, sem_si1, sem_di0, sem_di1, sem_xl0, sem_xl1,
                  sem_xr0, sem_xr1, sem_ew0, sem_ew1,
                  sem_sn0, sem_sn1, sem_sp0, sem_sp1):
    cid = lax.axis_index("c")
    sid = lax.axis_index("s")
    gid = cid * 16 + sid

    sis = (si0, si1)
    dis = (di0, di1)
    dscs = (dsc0, dsc1)
    pbs = (p0, p1)
    sems_sn = (sem_sn0, sem_sn1)
    sems_sp = (sem_sp0, sem_sp1)
    xls = (xl0, xl1)
    xrs = (xr0, xr1)
    ews = (ew0, ew1)
    sems_si = (sem_si0, sem_si1)
    sems_di = (sem_di0, sem_di1)
    sems_xl = (sem_xl0, sem_xl1)
    sems_xr = (sem_xr0, sem_xr1)
    sems_ew = (sem_ew0, sem_ew1)

    pltpu.sync_copy(att_hbm, att_v)

    def _zero_xl0(i, _):
        r = i // 8
        c = i % 8
        xl0[r, pl.ds(c * 16, 16)] = jnp.zeros((16,), f32)
        return 0
    lax.fori_loop(0, C * 8, _zero_xl0, 0)

    def _zero_zden(i, _):
        zden[pl.ds(i * 16, 16)] = jnp.zeros((16,), f32)
        return 0
    lax.fori_loop(0, RPT // 16, _zero_zden, 0)

    row0 = sid * RPT
    def _zero_acc(i, _):
        pltpu.sync_copy(xl0.at[pl.ds(0, SR), :],
                        num_acc.at[pl.ds(row0 + i * SR, SR), :])
        return 0
    lax.fori_loop(0, RPT // SR, _zero_acc, 0)
    pltpu.sync_copy(zden, den_acc.at[pl.ds(row0, RPT)])
    plsc.subcore_barrier()

    ebase = gid * EPT

    def _issue_idx(b, k):
        e0 = ebase + k * C
        pltpu.async_copy(src_hbm.at[pl.ds(e0, C)], sis[b], sems_si[b])
        pltpu.async_copy(dst_hbm.at[pl.ds(e0, C)], dis[b], sems_di[b])

    def _wait_idx(b):
        pltpu.make_async_copy(src_hbm.at[pl.ds(0, C)], sis[b],
                              sems_si[b]).wait()
        pltpu.make_async_copy(dst_hbm.at[pl.ds(0, C)], dis[b],
                              sems_di[b]).wait()

    def _issue_gathers(b, k):
        pltpu.async_copy(xl_hbm.at[sis[b]], xls[b], sems_xl[b])
        pltpu.async_copy(xr_hbm.at[dis[b]], xrs[b], sems_xr[b])
        pltpu.async_copy(ew_hbm.at[pl.ds(ebase + k * C, C)], ews[b],
                         sems_ew[b])

    def _wait_gathers(b):
        pltpu.make_async_copy(xl_hbm.at[sis[b]], xls[b],
                              sems_xl[b]).wait()
        pltpu.make_async_copy(xr_hbm.at[dis[b]], xrs[b],
                              sems_xr[b]).wait()
        pltpu.make_async_copy(ew_hbm.at[pl.ds(0, C)], ews[b],
                              sems_ew[b]).wait()

    def _save_dst(b, cnt=C):
        for j in range(cnt // 16):
            dscs[b][pl.ds(j * 16, 16)] = dis[b][pl.ds(j * 16, 16)]

    def _wait_scatters(b):
        pltpu.make_async_copy(xls[b], num_acc.at[dscs[b]],
                              sems_sn[b]).wait()
        pltpu.make_async_copy(pbs[b], den_acc.at[dscs[b]],
                              sems_sp[b]).wait()

    def _wait_scatters_rem(b, cnt):
        pltpu.make_async_copy(xls[b].at[pl.ds(0, cnt), :],
                              num_acc.at[dscs[b].at[pl.ds(0, cnt)]],
                              sems_sn[b]).wait()
        pltpu.make_async_copy(pbs[b].at[pl.ds(0, cnt)],
                              den_acc.at[dscs[b].at[pl.ds(0, cnt)]],
                              sems_sp[b]).wait()

    def _compute_scatter(b, cnt=C):
        xl_rows, xr_rows, ew_rows = xls[b], xrs[b], ews[b]

        att_c = tuple(att_v[pl.ds(c * 16, 16)] for c in range(8))
        lane = lax.iota(jnp.int32, 16)

        def _alpha_of(i):
            parts = []
            for c in range(8):
                sl = pl.ds(c * 16, 16)
                e = xl_rows[i, sl] + xr_rows[i, sl] + ew_rows[i, sl]
                e = jnp.maximum(e, 0.2 * e)
                parts.append(e * att_c[c])
            while len(parts) > 1:
                parts = [parts[2 * j] + parts[2 * j + 1]
                         for j in range(len(parts) // 2)]
            return jnp.sum(parts[0])

        def _edge_a(i2, carry):
            i = i2 * 2
            a_buf[i] = _alpha_of(i)
            a_buf[i + 1] = _alpha_of(i + 1)
            return carry
        lax.fori_loop(0, cnt // 2, _edge_a, 0)

        def _group(j, _):
            av = jnp.zeros((16,), f32)
            for l in range(16):
                av = jnp.where(lane == l, a_buf[j * 16 + l], av)
            pv16 = jnp.exp(av)
            pbs[b][pl.ds(j * 16, 16)] = pv16
            for l in range(16):
                i = j * 16 + l
                pv = jnp.full((16,), pv16[l], f32)
                for c in range(8):
                    sl = pl.ds(c * 16, 16)
                    xl_rows[i, sl] = xl_rows[i, sl] * pv
            return 0
        lax.fori_loop(0, cnt // 16, _group, 0)

        if cnt == C:
            pltpu.async_copy(xl_rows, num_acc.at[dscs[b]], sems_sn[b],
                             add=True)
            pltpu.async_copy(pbs[b], den_acc.at[dscs[b]], sems_sp[b],
                             add=True)
        else:
            pltpu.async_copy(xl_rows.at[pl.ds(0, cnt), :],
                             num_acc.at[dscs[b].at[pl.ds(0, cnt)]],
                             sems_sn[b], add=True)
            pltpu.async_copy(pbs[b].at[pl.ds(0, cnt)],
                             den_acc.at[dscs[b].at[pl.ds(0, cnt)]],
                             sems_sp[b], add=True)

    _issue_idx(0, 0)
    _wait_idx(0)
    _issue_gathers(0, 0)
    _issue_idx(1, 1)

    def _pair(k2, _):
        for par in (0, 1):
            k = k2 * 2 + par
            nb = 1 - par
            _wait_idx(nb)
            @pl.when(k >= 1)
            def _():
                _wait_scatters(nb)
            _issue_gathers(nb, k + 1)
            _wait_gathers(par)
            _save_dst(par)
            @pl.when(k + 2 < NCHUNK)
            def _():
                _issue_idx(par, k + 2)
            _compute_scatter(par)
        return 0

    lax.fori_loop(0, (NCHUNK - 2) // 2, _pair, 0)
    _wait_idx(1)
    _wait_scatters(1)
    _issue_gathers(1, NCHUNK - 1)
    _wait_gathers(0)
    _save_dst(0)
    _compute_scatter(0)
    _wait_gathers(1)
    _save_dst(1)
    _compute_scatter(1)

    _wait_scatters(0)
    e0 = ebase + NCHUNK * C
    pltpu.sync_copy(src_hbm.at[pl.ds(e0, REM)], si0.at[pl.ds(0, REM)])
    pltpu.sync_copy(dst_hbm.at[pl.ds(e0, REM)], di0.at[pl.ds(0, REM)])
    pltpu.async_copy(xl_hbm.at[si0.at[pl.ds(0, REM)]],
                     xl0.at[pl.ds(0, REM), :], sem_xl0).wait()
    pltpu.async_copy(xr_hbm.at[di0.at[pl.ds(0, REM)]],
                     xr0.at[pl.ds(0, REM), :], sem_xr0).wait()
    pltpu.async_copy(ew_hbm.at[pl.ds(e0, REM)],
                     ew0.at[pl.ds(0, REM), :], sem_ew0).wait()
    _save_dst(0, REM)
    _compute_scatter(0, REM)
    _wait_scatters_rem(0, REM)
    _wait_scatters(1)
    plsc.subcore_barrier()

    def _out(i, _):
        r = row0 + i * SR
        pltpu.sync_copy(num_acc.at[pl.ds(r, SR), :], xl0.at[pl.ds(0, SR), :])
        pltpu.sync_copy(xl0.at[pl.ds(0, SR), :], num_hbm.at[cid, pl.ds(r, SR), :])
        return 0
    lax.fori_loop(0, RPT // SR, _out, 0)
    pltpu.sync_copy(den_acc.at[pl.ds(row0, RPT)], zden)
    pltpu.sync_copy(zden, den_hbm.at[cid, pl.ds(row0, RPT)])


def _sc_edge_pass(xl, xr, ew, src, dst, att):
    mesh = plsc.VectorSubcoreMesh(core_axis_name="c", subcore_axis_name="s")
    kfn = pl.kernel(
        _sc_edge_body,
        out_type=(
            jax.ShapeDtypeStruct((2, NP, D), f32),
            jax.ShapeDtypeStruct((2, NP), f32),
        ),
        mesh=mesh,
        compiler_params=pltpu.CompilerParams(needs_layout_passes=False),
        scratch_types=[
            pltpu.VMEM((C,), jnp.int32),
            pltpu.VMEM((C,), jnp.int32),
            pltpu.VMEM((C,), jnp.int32),
            pltpu.VMEM((C,), jnp.int32),
            pltpu.VMEM((C,), jnp.int32),
            pltpu.VMEM((C,), jnp.int32),
            pltpu.VMEM((C, D), f32),
            pltpu.VMEM((C, D), f32),
            pltpu.VMEM((C, D), f32),
            pltpu.VMEM((C, D), f32),
            pltpu.VMEM((C, D), f32),
            pltpu.VMEM((C, D), f32),
            pltpu.SMEM((C,), f32),
            pltpu.VMEM((C,), f32),
            pltpu.VMEM((C,), f32),
            pltpu.VMEM((D,), f32),
            pltpu.VMEM((RPT,), f32),
            pltpu.VMEM_SHARED((NP, D), f32),
            pltpu.VMEM_SHARED((NP,), f32),
            pltpu.SemaphoreType.DMA,
            pltpu.SemaphoreType.DMA,
            pltpu.SemaphoreType.DMA,
            pltpu.SemaphoreType.DMA,
            pltpu.SemaphoreType.DMA,
            pltpu.SemaphoreType.DMA,
            pltpu.SemaphoreType.DMA,
            pltpu.SemaphoreType.DMA,
            pltpu.SemaphoreType.DMA,
            pltpu.SemaphoreType.DMA,
            pltpu.SemaphoreType.DMA,
            pltpu.SemaphoreType.DMA,
            pltpu.SemaphoreType.DMA,
            pltpu.SemaphoreType.DMA,
        ],
    )
    return kfn(xl, xr, ew, src, dst, att)


def kernel(x, edge_index, edge_attr,
           Wl1, Wr1, We1, att1, b1,
           Wl2, Wr2, We2, att2, b2,
           Wl3, Wr3, We3, att3, b3):
    src = edge_index[0]
    dst = edge_index[1]
    ew1 = _edge_mm(edge_attr, We1)
    ew2 = _edge_mm(edge_attr, We2)
    ew3 = _edge_mm(edge_attr, We3)
    xl, xr = _node_mm(x, Wl1, Wr1)
    num, den = _sc_edge_pass(xl, xr, ew1, src, dst, att1)
    xl, xr = _fin_mm(num, den, b1, Wl2, Wr2)
    num, den = _sc_edge_pass(xl, xr, ew2, src, dst, att2)
    xl, xr = _fin_mm(num, den, b2, Wl3, Wr3)
    num, den = _sc_edge_pass(xl, xr, ew3, src, dst, att3)
    return _finalize(num, den, b3, False)
